# baseline (speedup 1.0000x reference)
import functools

import jax
import jax.numpy as jnp
from jax import lax
from jax.experimental import pallas as pl
from jax.experimental.pallas import tpu as pltpu
from jax.experimental.pallas import tpu_sc as plsc

_N = 1024
_K = 64
_DIM = 128
_L = 2
_NREL = 32

_NC, _NS = 2, 16
_NW = _NC * _NS
_R = 2 * _N * _K * (1 + 2 * _L)
_PW = _R // _NW
_CH = 128
_NCHUNK = _PW // _CH

_BP = 64
_RB = _BP * _K


def _sc_gather(table, idx):
    mesh = plsc.VectorSubcoreMesh(
        core_axis_name="c", subcore_axis_name="s",
        num_cores=_NC, num_subcores=_NS)

    @functools.partial(
        pl.kernel,
        out_type=jax.ShapeDtypeStruct((_R, _DIM), jnp.float32),
        mesh=mesh,
        scratch_types=[
            pltpu.VMEM((4, _CH), jnp.int32),
            pltpu.VMEM((_CH, _DIM), jnp.float32),
            pltpu.VMEM((_CH, _DIM), jnp.float32),
            pltpu.VMEM((_CH, _DIM), jnp.float32),
            pltpu.VMEM((_CH, _DIM), jnp.float32),
            pltpu.SemaphoreType.DMA,
            pltpu.SemaphoreType.DMA,
            pltpu.SemaphoreType.DMA,
            pltpu.SemaphoreType.DMA,
            pltpu.SemaphoreType.DMA,
            pltpu.SemaphoreType.DMA,
            pltpu.SemaphoreType.DMA,
            pltpu.SemaphoreType.DMA,
            pltpu.SemaphoreType.DMA,
            pltpu.SemaphoreType.DMA,
            pltpu.SemaphoreType.DMA,
            pltpu.SemaphoreType.DMA,
        ],
    )
    def k(table_hbm, idx_hbm, out_hbm, idx_v, r0, r1, r2, r3,
          is0, is1, is2, is3, gs0, gs1, gs2, gs3, ss0, ss1, ss2, ss3):
        wid = lax.axis_index("s") * _NC + lax.axis_index("c")
        base = wid * _PW
        rows = (r0, r1, r2, r3)
        isem = (is0, is1, is2, is3)
        gsem = (gs0, gs1, gs2, gs3)
        ssem = (ss0, ss1, ss2, ss3)

        def istart(i, b):
            pltpu.async_copy(idx_hbm.at[pl.ds(base + i * _CH, _CH)],
                             idx_v.at[b], isem[b])

        def iwait(i, b):
            pltpu.make_async_copy(idx_hbm.at[pl.ds(base + i * _CH, _CH)],
                                  idx_v.at[b], isem[b]).wait()

        def gstart(b):
            pltpu.async_copy(table_hbm.at[idx_v.at[b]], rows[b], gsem[b])

        def gwait(b):
            pltpu.make_async_copy(table_hbm.at[idx_v.at[b]], rows[b],
                                  gsem[b]).wait()

        def sstart(i, b):
            pltpu.async_copy(rows[b],
                             out_hbm.at[pl.ds(base + i * _CH, _CH)], ssem[b])

        def swait(i, b):
            pltpu.make_async_copy(rows[b],
                                  out_hbm.at[pl.ds(base + i * _CH, _CH)],
                                  ssem[b]).wait()

        istart(0, 0)
        istart(1, 1)
        iwait(0, 0)
        gstart(0)
        istart(2, 2)
        iwait(1, 1)
        gstart(1)
        istart(3, 3)
        iwait(2, 2)
        gstart(2)
        gwait(0)
        sstart(0, 0)
        istart(4, 0)
        iwait(3, 3)
        gstart(3)
        gwait(1)
        sstart(1, 1)
        istart(5, 1)

        def quad(j, carry):
            i0 = 4 * j
            for b in range(4):
                i = i0 + b
                pb = (b + 2) % 4
                swait(i - 4, b)
                iwait(i, b)
                gstart(b)
                gwait(pb)
                sstart(i - 2, pb)
                istart(i + 2, pb)
            return carry

        lax.fori_loop(1, _NCHUNK // 4 - 1, quad, 0)

        for i in range(_NCHUNK - 4, _NCHUNK):
            b = i % 4
            pb = (b + 2) % 4
            swait(i - 4, b)
            iwait(i, b)
            gstart(b)
            gwait(pb)
            sstart(i - 2, pb)
            if i + 2 < _NCHUNK:
                istart(i + 2, pb)
        gwait((_NCHUNK - 2) % 4)
        sstart(_NCHUNK - 2, (_NCHUNK - 2) % 4)
        gwait((_NCHUNK - 1) % 4)
        sstart(_NCHUNK - 1, (_NCHUNK - 1) % 4)
        for i in range(_NCHUNK - 4, _NCHUNK):
            swait(i, i % 4)

    return k(table, idx)


def _attn_block(h_ref, t_ref, rel_ref, re_ref, w1_ref, b1_ref, w2_ref, b2_ref,
                w3_ref, b3_ref, out_ref):
    f32 = jnp.float32
    h = h_ref[0]
    t = t_ref[0]
    relv = rel_ref[0]
    w1a = w1_ref[0:_DIM, :]
    w1b = w1_ref[_DIM:2 * _DIM, :]
    rt = jnp.dot(re_ref[...], w1b, preferred_element_type=f32)
    cls = jax.lax.broadcasted_iota(jnp.int32, (_NREL, _RB), 0)
    oht = jnp.where(relv == cls, 1.0, 0.0)
    rb = jax.lax.dot_general(oht, rt, (((0,), (0,)), ((), ())),
                             preferred_element_type=f32)
    y1 = jnp.dot(h, w1a, preferred_element_type=f32) + rb + b1_ref[...]
    y1 = jnp.maximum(y1, 0.0)
    y2 = jnp.maximum(jnp.dot(y1, w2_ref[...], preferred_element_type=f32)
                     + b2_ref[...], 0.0)
    s = jax.nn.sigmoid(jnp.dot(y2, w3_ref[...], preferred_element_type=f32)
                       + b3_ref[...])
    e = jnp.exp(s[:, 0:1])
    num = jnp.sum((e * t).reshape(_BP, _K, _DIM), axis=1)
    den = jnp.sum(e.reshape(_BP, _K, 1), axis=1)
    out_ref[0] = num / den


def _mean_block(x_ref, out_ref):
    x = x_ref[0]
    out_ref[0] = jnp.sum(x.reshape(_BP, _K, _DIM), axis=1) * (1.0 / _K)


def _agg_block(emu_ref, pu0_ref, pu1_ref, emi_ref, pi0_ref, pi1_ref,
               wagg_ref, bagg_ref, out_ref):
    f32 = jnp.float32
    wg0 = wagg_ref[0:_DIM, :]
    wg1 = wagg_ref[_DIM:2 * _DIM, :]
    wg2 = wagg_ref[2 * _DIM:3 * _DIM, :]
    b = bagg_ref[...]
    ue = jax.nn.sigmoid(
        jnp.dot(emu_ref[...], wg0, preferred_element_type=f32)
        + jnp.dot(pu0_ref[...], wg1, preferred_element_type=f32)
        + jnp.dot(pu1_ref[...], wg2, preferred_element_type=f32) + b)
    ie = jax.nn.sigmoid(
        jnp.dot(emi_ref[...], wg0, preferred_element_type=f32)
        + jnp.dot(pi0_ref[...], wg1, preferred_element_type=f32)
        + jnp.dot(pi1_ref[...], wg2, preferred_element_type=f32) + b)
    out_ref[...] = jax.nn.sigmoid(jnp.sum(ue * ie, axis=1, keepdims=True))


def kernel(u_entity, u_heads, u_relations, u_tails,
           i_entity, i_heads, i_relations, i_tails,
           entity_emb, rel_emb, W1, b1, W2, b2, W3, b3, Wagg, bagg):
    f32 = jnp.float32
    i32 = jnp.int32
    nent_rows = 2 * _N * _K
    nhead_rows = 2 * _L * _N * _K

    idx = jnp.concatenate([
        u_entity.reshape(-1), i_entity.reshape(-1),
        u_heads.reshape(-1), i_heads.reshape(-1),
        u_tails.reshape(-1), i_tails.reshape(-1)]).astype(i32)

    ebf = entity_emb.astype(jnp.bfloat16)
    lo16 = jax.lax.bitcast_convert_type(ebf[:, :64], jnp.uint16).astype(i32)
    hi16 = jax.lax.bitcast_convert_type(ebf[:, 64:], jnp.uint16).astype(i32)
    tblp = jnp.bitwise_or(lo16, jnp.left_shift(hi16, 16))
    idx = idx + (tblp[0, 0] * 0)

    g = _sc_gather(entity_emb, idx)

    ent_rows = g[:nent_rows].reshape(-1, _RB, _DIM)
    head_rows = g[nent_rows:nent_rows + nhead_rows].reshape(-1, _RB, _DIM)
    tail_rows = g[nent_rows + nhead_rows:].reshape(-1, _RB, _DIM)

    rel = jnp.concatenate([u_relations, i_relations], axis=0)
    rel = rel.reshape(-1, 1, _RB).astype(i32)

    w3p = jnp.pad(W3, ((0, 0), (0, _DIM - 1)))
    b3p = jnp.pad(b3.reshape(1, 1), ((0, 0), (0, _DIM - 1)))
    nblk = head_rows.shape[0]

    pooled = pl.pallas_call(
        _attn_block,
        grid=(nblk,),
        in_specs=[
            pl.BlockSpec((1, _RB, _DIM), lambda i: (i, 0, 0)),
            pl.BlockSpec((1, _RB, _DIM), lambda i: (i, 0, 0)),
            pl.BlockSpec((1, 1, _RB), lambda i: (i, 0, 0)),
            pl.BlockSpec((_NREL, _DIM), lambda i: (0, 0)),
            pl.BlockSpec((2 * _DIM, _DIM), lambda i: (0, 0)),
            pl.BlockSpec((1, _DIM), lambda i: (0, 0)),
            pl.BlockSpec((_DIM, _DIM), lambda i: (0, 0)),
            pl.BlockSpec((1, _DIM), lambda i: (0, 0)),
            pl.BlockSpec((_DIM, _DIM), lambda i: (0, 0)),
            pl.BlockSpec((1, _DIM), lambda i: (0, 0)),
        ],
        out_specs=pl.BlockSpec((1, _BP, _DIM), lambda i: (i, 0, 0)),
        out_shape=jax.ShapeDtypeStruct((nblk, _BP, _DIM), f32),
    )(head_rows, tail_rows, rel, rel_emb, W1, b1.reshape(1, _DIM), W2,
      b2.reshape(1, _DIM), w3p, b3p)

    eblk = ent_rows.shape[0]
    means = pl.pallas_call(
        _mean_block,
        grid=(eblk,),
        in_specs=[pl.BlockSpec((1, _RB, _DIM), lambda i: (i, 0, 0))],
        out_specs=pl.BlockSpec((1, _BP, _DIM), lambda i: (i, 0, 0)),
        out_shape=jax.ShapeDtypeStruct((eblk, _BP, _DIM), f32),
    )(ent_rows)

    means = means.reshape(2, _N, _DIM)
    pooled = pooled.reshape(2 * _L, _N, _DIM)

    out = pl.pallas_call(
        _agg_block,
        in_specs=[pl.BlockSpec((_N, _DIM), lambda: (0, 0))] * 6
        + [pl.BlockSpec(((_L + 1) * _DIM, _DIM), lambda: (0, 0)),
           pl.BlockSpec((1, _DIM), lambda: (0, 0))],
        out_specs=pl.BlockSpec((_N, 1), lambda: (0, 0)),
        out_shape=jax.ShapeDtypeStruct((_N, 1), f32),
    )(means[0], pooled[0], pooled[1], means[1], pooled[2], pooled[3],
      Wagg, bagg.reshape(1, _DIM))

    return out.reshape(_N)


# speedup vs baseline: 1.1794x; 1.1794x over previous
import functools

import jax
import jax.numpy as jnp
from jax import lax
from jax.experimental import pallas as pl
from jax.experimental.pallas import tpu as pltpu
from jax.experimental.pallas import tpu_sc as plsc

_N = 1024
_K = 64
_DIM = 128
_L = 2
_NREL = 32

_NC, _NS = 2, 16
_NW = _NC * _NS
_R = 2 * _N * _K * 2 * _L
_PW = _R // _NW
_CH = 128
_NCHUNK = _PW // _CH

_BP = 64
_RB = _BP * _K


def _sc_gather(table, idx):
    mesh = plsc.VectorSubcoreMesh(
        core_axis_name="c", subcore_axis_name="s",
        num_cores=_NC, num_subcores=_NS)

    @functools.partial(
        pl.kernel,
        out_type=jax.ShapeDtypeStruct((_R, _DIM), jnp.float32),
        mesh=mesh,
        scratch_types=[
            pltpu.VMEM((4, _CH), jnp.int32),
            pltpu.VMEM((_CH, _DIM), jnp.float32),
            pltpu.VMEM((_CH, _DIM), jnp.float32),
            pltpu.VMEM((_CH, _DIM), jnp.float32),
            pltpu.VMEM((_CH, _DIM), jnp.float32),
            pltpu.SemaphoreType.DMA,
            pltpu.SemaphoreType.DMA,
            pltpu.SemaphoreType.DMA,
            pltpu.SemaphoreType.DMA,
            pltpu.SemaphoreType.DMA,
            pltpu.SemaphoreType.DMA,
            pltpu.SemaphoreType.DMA,
            pltpu.SemaphoreType.DMA,
            pltpu.SemaphoreType.DMA,
            pltpu.SemaphoreType.DMA,
            pltpu.SemaphoreType.DMA,
            pltpu.SemaphoreType.DMA,
        ],
    )
    def k(table_hbm, idx_hbm, out_hbm, idx_v, r0, r1, r2, r3,
          is0, is1, is2, is3, gs0, gs1, gs2, gs3, ss0, ss1, ss2, ss3):
        wid = lax.axis_index("s") * _NC + lax.axis_index("c")
        base = wid * _PW
        rows = (r0, r1, r2, r3)
        isem = (is0, is1, is2, is3)
        gsem = (gs0, gs1, gs2, gs3)
        ssem = (ss0, ss1, ss2, ss3)

        def istart(i, b):
            pltpu.async_copy(idx_hbm.at[pl.ds(base + i * _CH, _CH)],
                             idx_v.at[b], isem[b])

        def iwait(i, b):
            pltpu.make_async_copy(idx_hbm.at[pl.ds(base + i * _CH, _CH)],
                                  idx_v.at[b], isem[b]).wait()

        def gstart(b):
            pltpu.async_copy(table_hbm.at[idx_v.at[b]], rows[b], gsem[b])

        def gwait(b):
            pltpu.make_async_copy(table_hbm.at[idx_v.at[b]], rows[b],
                                  gsem[b]).wait()

        def sstart(i, b):
            pltpu.async_copy(rows[b],
                             out_hbm.at[pl.ds(base + i * _CH, _CH)], ssem[b])

        def swait(i, b):
            pltpu.make_async_copy(rows[b],
                                  out_hbm.at[pl.ds(base + i * _CH, _CH)],
                                  ssem[b]).wait()

        istart(0, 0)
        istart(1, 1)
        iwait(0, 0)
        gstart(0)
        istart(2, 2)
        iwait(1, 1)
        gstart(1)
        istart(3, 3)
        iwait(2, 2)
        gstart(2)
        gwait(0)
        sstart(0, 0)
        istart(4, 0)
        iwait(3, 3)
        gstart(3)
        gwait(1)
        sstart(1, 1)
        istart(5, 1)

        def quad(j, carry):
            i0 = 4 * j
            for b in range(4):
                i = i0 + b
                pb = (b + 2) % 4
                swait(i - 4, b)
                iwait(i, b)
                gstart(b)
                gwait(pb)
                sstart(i - 2, pb)
                istart(i + 2, pb)
            return carry

        lax.fori_loop(1, _NCHUNK // 4 - 1, quad, 0)

        for i in range(_NCHUNK - 4, _NCHUNK):
            b = i % 4
            pb = (b + 2) % 4
            swait(i - 4, b)
            iwait(i, b)
            gstart(b)
            gwait(pb)
            sstart(i - 2, pb)
            if i + 2 < _NCHUNK:
                istart(i + 2, pb)
        gwait((_NCHUNK - 2) % 4)
        sstart(_NCHUNK - 2, (_NCHUNK - 2) % 4)
        gwait((_NCHUNK - 1) % 4)
        sstart(_NCHUNK - 1, (_NCHUNK - 1) % 4)
        for i in range(_NCHUNK - 4, _NCHUNK):
            swait(i, i % 4)

    return k(table, idx)


_NE = 2 * _N * _K
_EPW = _NE // _NW
_ECH = _NE // _NW // _CH


def _sc_entity_mean(table, idx):
    mesh = plsc.VectorSubcoreMesh(
        core_axis_name="c", subcore_axis_name="s",
        num_cores=_NC, num_subcores=_NS)

    @functools.partial(
        pl.kernel,
        out_type=jax.ShapeDtypeStruct((_NE // _K, _DIM), jnp.float32),
        mesh=mesh,
        scratch_types=[
            pltpu.VMEM((2, _CH), jnp.int32),
            pltpu.VMEM((_CH, _DIM), jnp.float32),
            pltpu.VMEM((_CH, _DIM), jnp.float32),
            pltpu.VMEM((2 * _ECH, _DIM), jnp.float32),
            pltpu.SemaphoreType.DMA,
            pltpu.SemaphoreType.DMA,
            pltpu.SemaphoreType.DMA,
            pltpu.SemaphoreType.DMA,
        ],
    )
    def k(table_hbm, idx_hbm, out_hbm, idx_v, r0, r1, out_v,
          is0, is1, gs0, gs1):
        wid = lax.axis_index("s") * _NC + lax.axis_index("c")
        base = wid * _EPW
        rows = (r0, r1)
        isem = (is0, is1)
        gsem = (gs0, gs1)

        def istart(i, b):
            pltpu.async_copy(idx_hbm.at[pl.ds(base + i * _CH, _CH)],
                             idx_v.at[b], isem[b])

        def iwait(i, b):
            pltpu.make_async_copy(idx_hbm.at[pl.ds(base + i * _CH, _CH)],
                                  idx_v.at[b], isem[b]).wait()

        def gstart(b):
            pltpu.async_copy(table_hbm.at[idx_v.at[b]], rows[b], gsem[b])

        def gwait(b):
            pltpu.make_async_copy(table_hbm.at[idx_v.at[b]], rows[b],
                                  gsem[b]).wait()

        def accum(b, orow):
            for g2 in range(2):
                def body(r, acc):
                    return tuple(
                        acc[c] + rows[b][g2 * _K + r, pl.ds(16 * c, 16)]
                        for c in range(8))
                acc = lax.fori_loop(
                    0, _K, body,
                    tuple(jnp.zeros((16,), jnp.float32) for _ in range(8)))
                for c in range(8):
                    out_v[orow + g2, pl.ds(16 * c, 16)] = acc[c] * (1.0 / _K)

        istart(0, 0)
        istart(1, 1)
        iwait(0, 0)
        gstart(0)

        def pair(j, carry):
            i0 = 2 * j
            i1 = i0 + 1
            iwait(i1, 1)
            gstart(1)
            gwait(0)

            @pl.when(j < _ECH // 2 - 1)
            def _pre0():
                istart(i0 + 2, 0)

            accum(0, 2 * i0)

            @pl.when(j < _ECH // 2 - 1)
            def _go0():
                iwait(i0 + 2, 0)
                gstart(0)

            gwait(1)

            @pl.when(j < _ECH // 2 - 1)
            def _pre1():
                istart(i1 + 2, 1)

            accum(1, 2 * i1)
            return carry

        lax.fori_loop(0, _ECH // 2, pair, 0)
        pltpu.sync_copy(out_v, out_hbm.at[pl.ds(wid * 2 * _ECH, 2 * _ECH)])

    return k(table, idx)


def _attn_block(h_ref, t_ref, rel_ref, re_ref, w1_ref, b1_ref, w2_ref, b2_ref,
                w3_ref, b3_ref, out_ref):
    f32 = jnp.float32
    h = h_ref[0]
    t = t_ref[0]
    relv = rel_ref[0]
    w1a = w1_ref[0:_DIM, :]
    w1b = w1_ref[_DIM:2 * _DIM, :]
    rt = jnp.dot(re_ref[...], w1b, preferred_element_type=f32)
    cls = jax.lax.broadcasted_iota(jnp.int32, (_NREL, _RB), 0)
    oht = jnp.where(relv == cls, 1.0, 0.0)
    rb = jax.lax.dot_general(oht, rt, (((0,), (0,)), ((), ())),
                             preferred_element_type=f32)
    y1 = jnp.dot(h, w1a, preferred_element_type=f32) + rb + b1_ref[...]
    y1 = jnp.maximum(y1, 0.0)
    y2 = jnp.maximum(jnp.dot(y1, w2_ref[...], preferred_element_type=f32)
                     + b2_ref[...], 0.0)
    s = jax.nn.sigmoid(jnp.dot(y2, w3_ref[...], preferred_element_type=f32)
                       + b3_ref[...])
    e = jnp.exp(s[:, 0:1])
    num = jnp.sum((e * t).reshape(_BP, _K, _DIM), axis=1)
    den = jnp.sum(e.reshape(_BP, _K, 1), axis=1)
    out_ref[0] = num / den


def _agg_block(emu_ref, pu0_ref, pu1_ref, emi_ref, pi0_ref, pi1_ref,
               wagg_ref, bagg_ref, out_ref):
    f32 = jnp.float32
    wg0 = wagg_ref[0:_DIM, :]
    wg1 = wagg_ref[_DIM:2 * _DIM, :]
    wg2 = wagg_ref[2 * _DIM:3 * _DIM, :]
    b = bagg_ref[...]
    ue = jax.nn.sigmoid(
        jnp.dot(emu_ref[...], wg0, preferred_element_type=f32)
        + jnp.dot(pu0_ref[...], wg1, preferred_element_type=f32)
        + jnp.dot(pu1_ref[...], wg2, preferred_element_type=f32) + b)
    ie = jax.nn.sigmoid(
        jnp.dot(emi_ref[...], wg0, preferred_element_type=f32)
        + jnp.dot(pi0_ref[...], wg1, preferred_element_type=f32)
        + jnp.dot(pi1_ref[...], wg2, preferred_element_type=f32) + b)
    out_ref[...] = jax.nn.sigmoid(jnp.sum(ue * ie, axis=1, keepdims=True))


def kernel(u_entity, u_heads, u_relations, u_tails,
           i_entity, i_heads, i_relations, i_tails,
           entity_emb, rel_emb, W1, b1, W2, b2, W3, b3, Wagg, bagg):
    f32 = jnp.float32
    i32 = jnp.int32
    nhead_rows = 2 * _L * _N * _K

    idx_ent = jnp.concatenate([
        u_entity.reshape(-1), i_entity.reshape(-1)]).astype(i32)
    idx = jnp.concatenate([
        u_heads.reshape(-1), i_heads.reshape(-1),
        u_tails.reshape(-1), i_tails.reshape(-1)]).astype(i32)

    means = _sc_entity_mean(entity_emb, idx_ent)
    g = _sc_gather(entity_emb, idx)

    head_rows = g[:nhead_rows].reshape(-1, _RB, _DIM)
    tail_rows = g[nhead_rows:].reshape(-1, _RB, _DIM)

    rel = jnp.concatenate([u_relations, i_relations], axis=0)
    rel = rel.reshape(-1, 1, _RB).astype(i32)

    w3p = jnp.pad(W3, ((0, 0), (0, _DIM - 1)))
    b3p = jnp.pad(b3.reshape(1, 1), ((0, 0), (0, _DIM - 1)))
    nblk = head_rows.shape[0]

    pooled = pl.pallas_call(
        _attn_block,
        grid=(nblk,),
        in_specs=[
            pl.BlockSpec((1, _RB, _DIM), lambda i: (i, 0, 0)),
            pl.BlockSpec((1, _RB, _DIM), lambda i: (i, 0, 0)),
            pl.BlockSpec((1, 1, _RB), lambda i: (i, 0, 0)),
            pl.BlockSpec((_NREL, _DIM), lambda i: (0, 0)),
            pl.BlockSpec((2 * _DIM, _DIM), lambda i: (0, 0)),
            pl.BlockSpec((1, _DIM), lambda i: (0, 0)),
            pl.BlockSpec((_DIM, _DIM), lambda i: (0, 0)),
            pl.BlockSpec((1, _DIM), lambda i: (0, 0)),
            pl.BlockSpec((_DIM, _DIM), lambda i: (0, 0)),
            pl.BlockSpec((1, _DIM), lambda i: (0, 0)),
        ],
        out_specs=pl.BlockSpec((1, _BP, _DIM), lambda i: (i, 0, 0)),
        out_shape=jax.ShapeDtypeStruct((nblk, _BP, _DIM), f32),
    )(head_rows, tail_rows, rel, rel_emb, W1, b1.reshape(1, _DIM), W2,
      b2.reshape(1, _DIM), w3p, b3p)

    means = means.reshape(2, _N, _DIM)
    pooled = pooled.reshape(2 * _L, _N, _DIM)

    out = pl.pallas_call(
        _agg_block,
        in_specs=[pl.BlockSpec((_N, _DIM), lambda: (0, 0))] * 6
        + [pl.BlockSpec(((_L + 1) * _DIM, _DIM), lambda: (0, 0)),
           pl.BlockSpec((1, _DIM), lambda: (0, 0))],
        out_specs=pl.BlockSpec((_N, 1), lambda: (0, 0)),
        out_shape=jax.ShapeDtypeStruct((_N, 1), f32),
    )(means[0], pooled[0], pooled[1], means[1], pooled[2], pooled[3],
      Wagg, bagg.reshape(1, _DIM))

    return out.reshape(_N)
